# baseline (speedup 1.0000x reference)
import functools

import jax
import jax.numpy as jnp
from jax import lax
from jax.experimental import pallas as pl
from jax.experimental.pallas import tpu as pltpu
from jax.experimental.pallas import tpu_sc as plsc

H = 768
L = 16
NSL = H // L
B = 1024
S = 50
EPS = 1e-12
NC = 2
NS = 16
NW = NC * NS
BPW = B // NW


def _rsqrt_vec(v):
    i = plsc.bitcast(v, jnp.int32)
    i = jnp.full((L,), 0x5F3759DF, jnp.int32) - lax.shift_right_logical(i, 1)
    y = plsc.bitcast(i, jnp.float32)
    half = v * 0.5
    for _ in range(3):
        y = y * (1.5 - half * y * y)
    return y


def _row_stats(buf, r):
    def body(j, acc):
        s, q = acc
        x = buf[r, pl.ds(j * L, L)]
        return (s + x, q + x * x)

    z = jnp.zeros((L,), jnp.float32)
    s, q = lax.fori_loop(0, NSL, body, (z, z), unroll=4)
    tot = jnp.sum(s)
    totq = jnp.sum(q)
    mean = tot * (1.0 / H)
    var = totq * (1.0 / H) - mean * mean
    rstd = _rsqrt_vec(jnp.full((L,), var + EPS, jnp.float32))
    return jnp.full((L,), mean, jnp.float32), rstd


def _ln_pos_row(pos_v, r, eg_v, eb_v, ab_v):
    mean, rstd = _row_stats(pos_v, r)

    def body(j, carry):
        ds = pl.ds(j * L, L)
        x = pos_v[r, ds]
        pos_v[r, ds] = (x - mean) * rstd * eg_v[ds] + eb_v[ds] + ab_v[ds]
        return carry

    lax.fori_loop(0, NSL, body, 0, unroll=4)


def _ln_row_add(buf, r, ag_v, pos_v):
    mean, rstd = _row_stats(buf, r)

    def body(j, carry):
        ds = pl.ds(j * L, L)
        x = buf[r, ds]
        buf[r, ds] = (x - mean) * rstd * ag_v[ds] + pos_v[r, ds]
        return carry

    lax.fori_loop(0, NSL, body, 0, unroll=4)


def _process_chunk(buf, ag_v, pos_v):
    def body(r, carry):
        _ln_row_add(buf, r, ag_v, pos_v)
        return carry

    lax.fori_loop(0, S, body, 0)


_mesh = plsc.VectorSubcoreMesh(
    core_axis_name="c", subcore_axis_name="s", num_cores=NC, num_subcores=NS
)


@functools.partial(
    pl.kernel,
    out_type=jax.ShapeDtypeStruct((B * S, H), jnp.float32),
    mesh=_mesh,
    scratch_types=[
        pltpu.VMEM((BPW, S), jnp.int32),
        pltpu.VMEM((S, H), jnp.float32),
        pltpu.VMEM((S, H), jnp.float32),
        pltpu.VMEM((S, H), jnp.float32),
        pltpu.VMEM((H,), jnp.float32),
        pltpu.VMEM((H,), jnp.float32),
        pltpu.VMEM((H,), jnp.float32),
        pltpu.VMEM((H,), jnp.float32),
        pltpu.SemaphoreType.DMA,
        pltpu.SemaphoreType.DMA,
        pltpu.SemaphoreType.DMA,
        pltpu.SemaphoreType.DMA,
    ],
    compiler_params=pltpu.CompilerParams(
        use_tc_tiling_on_sc=False, needs_layout_passes=False
    ),
)
def _sc_kernel(ans_hbm, idx_hbm, pos_hbm, ag_hbm, ab_hbm, eg_hbm, eb_hbm,
               out_hbm, idx_v, pos_v, buf0, buf1, ag_v, ab_v, eg_v, eb_v,
               gsem0, gsem1, osem0, osem1):
    wid = lax.axis_index("s") * NC + lax.axis_index("c")
    b0 = wid * BPW

    pltpu.sync_copy(idx_hbm.at[pl.ds(b0, BPW)], idx_v)
    pltpu.sync_copy(pos_hbm.at[pl.ds(0, S)], pos_v)
    pltpu.sync_copy(ag_hbm, ag_v)
    pltpu.sync_copy(ab_hbm, ab_v)
    pltpu.sync_copy(eg_hbm, eg_v)
    pltpu.sync_copy(eb_hbm, eb_v)

    def gather_start(n, buf, sem):
        pltpu.make_async_copy(ans_hbm.at[idx_v.at[n]], buf, sem).start()

    def gather_wait(n, buf, sem):
        pltpu.make_async_copy(ans_hbm.at[idx_v.at[n]], buf, sem).wait()

    def out_start(n, buf, sem):
        pltpu.make_async_copy(buf, out_hbm.at[pl.ds((b0 + n) * S, S)], sem).start()

    def out_wait(n, buf, sem):
        pltpu.make_async_copy(buf, out_hbm.at[pl.ds((b0 + n) * S, S)], sem).wait()

    gather_start(0, buf0, gsem0)
    gather_start(1, buf1, gsem1)

    def posbody(r, carry):
        _ln_pos_row(pos_v, r, eg_v, eb_v, ab_v)
        return carry

    lax.fori_loop(0, S, posbody, 0)

    def pair(i, carry):
        n0 = 2 * i
        gather_wait(n0, buf0, gsem0)
        _process_chunk(buf0, ag_v, pos_v)
        out_start(n0, buf0, osem0)

        gather_wait(n0 + 1, buf1, gsem1)
        _process_chunk(buf1, ag_v, pos_v)
        out_start(n0 + 1, buf1, osem1)

        out_wait(n0, buf0, osem0)
        gather_start(n0 + 2, buf0, gsem0)
        out_wait(n0 + 1, buf1, osem1)
        gather_start(n0 + 3, buf1, gsem1)
        return carry

    lax.fori_loop(0, BPW // 2 - 1, pair, 0)

    n0 = BPW - 2
    gather_wait(n0, buf0, gsem0)
    _process_chunk(buf0, ag_v, pos_v)
    out_start(n0, buf0, osem0)

    gather_wait(n0 + 1, buf1, gsem1)
    _process_chunk(buf1, ag_v, pos_v)
    out_start(n0 + 1, buf1, osem1)

    out_wait(n0, buf0, osem0)
    out_wait(n0 + 1, buf1, osem1)


def kernel(ans_emb, prev_inds, pos_table, ans_ln_g, ans_ln_b, emb_ln_g, emb_ln_b):
    out = _sc_kernel(
        ans_emb,
        prev_inds.astype(jnp.int32),
        pos_table,
        ans_ln_g,
        ans_ln_b,
        emb_ln_g,
        emb_ln_b,
    )
    return out.reshape(B, S, H)
